# baseline (speedup 1.0000x reference)
import functools

import jax
import jax.numpy as jnp
from jax import lax
from jax.experimental import pallas as pl
from jax.experimental.pallas import tpu as pltpu
from jax.experimental.pallas import tpu_sc as plsc

B = 16384
F = 26
V = 100000
D = 32
C = 13
H1 = 128
H2 = 64
NCLS = 2

XB = 8192
NXB = 13
VPAD = NXB * XB

NW = 32
PER_W = (B * F) // NW
IDX_ROW = 128
ROWS_PER_W = PER_W // IDX_ROW
ROWS_PER_CHUNK = 4
NCHUNK = ROWS_PER_W // ROWS_PER_CHUNK
CHUNK = ROWS_PER_CHUNK * IDX_ROW


def _tr_body(in_ref, eye_ref, out_ref):
    h = in_ref[...].reshape(D, XB)
    ht = lax.dot_general(h, eye_ref[...], (((0,), (0,)), ((), ())),
                         preferred_element_type=jnp.float32)
    q = XB // 4
    out_ref[...] = jnp.concatenate(
        [ht[k * q:(k + 1) * q] for k in range(4)], axis=1)


@jax.jit
def _tc_transpose(table_t):
    eye = jnp.eye(D, dtype=jnp.float32)
    return pl.pallas_call(
        _tr_body,
        grid=(F, NXB),
        in_specs=[pl.BlockSpec((1, D, XB), lambda f, b: (f, 0, b)),
                  pl.BlockSpec((D, D), lambda f, b: (0, 0))],
        out_specs=pl.BlockSpec((XB // 4, 128),
                               lambda f, b: (f * NXB + b, 0)),
        out_shape=jax.ShapeDtypeStruct((F * VPAD // 4, 128), jnp.float32),
    )(table_t, eye)


def _gather_body(table_hbm, idx_hbm, out_hbm, idx_v, rows_v, sem):
    c = lax.axis_index("c")
    s = lax.axis_index("s")
    wid = s * 2 + c
    pltpu.sync_copy(idx_hbm.at[pl.ds(wid * ROWS_PER_W, ROWS_PER_W)], idx_v)
    base = wid * PER_W

    def chunk_body(ci, _):
        r0 = ci * ROWS_PER_CHUNK
        for j in range(ROWS_PER_CHUNK):
            pltpu.async_copy(
                table_hbm.at[idx_v.at[r0 + j]],
                rows_v.at[pl.ds(j * IDX_ROW, IDX_ROW)],
                sem,
            )
        for j in range(ROWS_PER_CHUNK):
            pltpu.make_async_copy(
                table_hbm.at[idx_v.at[r0 + j]],
                rows_v.at[pl.ds(j * IDX_ROW, IDX_ROW)],
                sem,
            ).wait()
        pltpu.sync_copy(rows_v, out_hbm.at[pl.ds(base + ci * CHUNK, CHUNK)])
        return 0

    lax.fori_loop(0, NCHUNK, chunk_body, 0)


@jax.jit
def _sc_gather(table_rows, idx2d):
    mesh = plsc.VectorSubcoreMesh(core_axis_name="c", subcore_axis_name="s")
    return pl.kernel(
        _gather_body,
        out_type=jax.ShapeDtypeStruct((B * F, D), jnp.float32),
        mesh=mesh,
        scratch_types=[
            pltpu.VMEM((ROWS_PER_W, IDX_ROW), jnp.int32),
            pltpu.VMEM((CHUNK, D), jnp.float32),
            pltpu.SemaphoreType.DMA,
        ],
        compiler_params=pltpu.CompilerParams(use_tc_tiling_on_sc=False),
    )(table_rows, idx2d)


def _mlp_body(emb_ref, xc_ref, w1a_ref, w1b_ref, b1_ref, w2_ref, b2_ref,
              w3_ref, b3_ref, out_ref):
    h = jnp.dot(emb_ref[...], w1a_ref[...], preferred_element_type=jnp.float32)
    h = h + jnp.dot(xc_ref[...], w1b_ref[...],
                    preferred_element_type=jnp.float32)
    h = jnp.maximum(h + b1_ref[...], 0.0)
    h = jnp.dot(h, w2_ref[...], preferred_element_type=jnp.float32)
    h = jnp.maximum(h + b2_ref[...], 0.0)
    o = jnp.dot(h, w3_ref[...], preferred_element_type=jnp.float32)
    out_ref[...] = o + b3_ref[...]


BM = 1024


@jax.jit
def _mlp(emb, xc_pad, w1a, w1b, b1, w2p, b2p, w3p, b3p):
    grid = (B // BM,)
    return pl.pallas_call(
        _mlp_body,
        grid=grid,
        in_specs=[
            pl.BlockSpec((BM, F * D), lambda i: (i, 0)),
            pl.BlockSpec((BM, 128), lambda i: (i, 0)),
            pl.BlockSpec((F * D, H1), lambda i: (0, 0)),
            pl.BlockSpec((128, H1), lambda i: (0, 0)),
            pl.BlockSpec((1, H1), lambda i: (0, 0)),
            pl.BlockSpec((H1, 128), lambda i: (0, 0)),
            pl.BlockSpec((1, 128), lambda i: (0, 0)),
            pl.BlockSpec((128, 128), lambda i: (0, 0)),
            pl.BlockSpec((1, 128), lambda i: (0, 0)),
        ],
        out_specs=pl.BlockSpec((BM, 128), lambda i: (i, 0)),
        out_shape=jax.ShapeDtypeStruct((B, 128), jnp.float32),
    )(emb, xc_pad, w1a, w1b, b1, w2p, b2p, w3p, b3p)


def kernel(x_cat, x_cont, tables, W1, b1, W2, b2, W3, b3):
    table_t = tables.transpose(0, 2, 1)
    table_rows = _tc_transpose(table_t).reshape(F * VPAD, D)

    x = x_cat.astype(jnp.int32)
    fofs = (jnp.arange(F, dtype=jnp.int32) * VPAD)[None, :]
    blk = x // XB
    loc = x % XB
    q = loc % (XB // 4)
    k = loc // (XB // 4)
    flat_idx = fofs + blk * XB + (q * 4 + k)
    idx2d = flat_idx.reshape((B * F) // IDX_ROW, IDX_ROW)

    emb = _sc_gather(table_rows, idx2d).reshape(B, F * D)

    xc_pad = jnp.pad(x_cont, ((0, 0), (0, 128 - C)))
    w1a = W1[:F * D]
    w1b = jnp.pad(W1[F * D:], ((0, 128 - C), (0, 0)))
    w2p = jnp.pad(W2, ((0, 0), (0, 128 - H2)))
    b2p = jnp.pad(b2, (0, 128 - H2)).reshape(1, 128)
    w3p = jnp.pad(W3, ((0, 128 - H2), (0, 128 - NCLS)))
    b3p = jnp.pad(b3, (0, 128 - NCLS)).reshape(1, 128)

    out = _mlp(emb, xc_pad, w1a, w1b, b1.reshape(1, H1), w2p, b2p, w3p, b3p)
    return out[:, :NCLS]


# speedup vs baseline: 1.5561x; 1.5561x over previous
import functools

import jax
import jax.numpy as jnp
from jax import lax
from jax.experimental import pallas as pl
from jax.experimental.pallas import tpu as pltpu
from jax.experimental.pallas import tpu_sc as plsc

B = 16384
F = 26
V = 100000
D = 32
C = 13
H1 = 128
H2 = 64
NCLS = 2

XB = 2048
NXB = 49
VPAD = NXB * XB
FB = (F + 3) // 4

NW = 32
PER_W = (B * F) // NW
IDX_ROW = 128
ROWS_PER_W = PER_W // IDX_ROW
ROWS_PER_CHUNK = 4
NCHUNK = ROWS_PER_W // ROWS_PER_CHUNK
CHUNK = ROWS_PER_CHUNK * IDX_ROW


def _tr_body(in_ref, out_ref):
    h = in_ref[...].reshape(4 * D, XB)
    out_ref[...] = h.T


@jax.jit
def _tc_transpose(table_t):
    return pl.pallas_call(
        _tr_body,
        grid=(FB, NXB),
        in_specs=[pl.BlockSpec((4, D, XB), lambda fb, b: (fb, 0, b))],
        out_specs=pl.BlockSpec((XB, 128),
                               lambda fb, b: (fb * NXB + b, 0)),
        out_shape=jax.ShapeDtypeStruct((FB * NXB * XB, 128), jnp.float32),
    )(table_t)


def _gather_body(table_hbm, idx_hbm, out_hbm, idx_v, rows_v, sem):
    c = lax.axis_index("c")
    s = lax.axis_index("s")
    wid = s * 2 + c
    pltpu.sync_copy(idx_hbm.at[pl.ds(wid * ROWS_PER_W, ROWS_PER_W)], idx_v)
    base = wid * PER_W

    def chunk_body(ci, _):
        r0 = ci * ROWS_PER_CHUNK
        for j in range(ROWS_PER_CHUNK):
            pltpu.async_copy(
                table_hbm.at[idx_v.at[r0 + j]],
                rows_v.at[pl.ds(j * IDX_ROW, IDX_ROW)],
                sem,
            )
        for j in range(ROWS_PER_CHUNK):
            pltpu.make_async_copy(
                table_hbm.at[idx_v.at[r0 + j]],
                rows_v.at[pl.ds(j * IDX_ROW, IDX_ROW)],
                sem,
            ).wait()
        pltpu.sync_copy(rows_v, out_hbm.at[pl.ds(base + ci * CHUNK, CHUNK)])
        return 0

    lax.fori_loop(0, NCHUNK, chunk_body, 0)


@jax.jit
def _sc_gather(table_rows, idx2d):
    mesh = plsc.VectorSubcoreMesh(core_axis_name="c", subcore_axis_name="s")
    return pl.kernel(
        _gather_body,
        out_type=jax.ShapeDtypeStruct((B * F, D), jnp.float32),
        mesh=mesh,
        scratch_types=[
            pltpu.VMEM((ROWS_PER_W, IDX_ROW), jnp.int32),
            pltpu.VMEM((CHUNK, D), jnp.float32),
            pltpu.SemaphoreType.DMA,
        ],
        compiler_params=pltpu.CompilerParams(use_tc_tiling_on_sc=False),
    )(table_rows, idx2d)


def _mlp_body(emb_ref, xc_ref, w1a_ref, w1b_ref, b1_ref, w2_ref, b2_ref,
              w3_ref, b3_ref, out_ref):
    h = jnp.dot(emb_ref[...], w1a_ref[...], preferred_element_type=jnp.float32)
    h = h + jnp.dot(xc_ref[...], w1b_ref[...],
                    preferred_element_type=jnp.float32)
    h = jnp.maximum(h + b1_ref[...], 0.0)
    h = jnp.dot(h, w2_ref[...], preferred_element_type=jnp.float32)
    h = jnp.maximum(h + b2_ref[...], 0.0)
    o = jnp.dot(h, w3_ref[...], preferred_element_type=jnp.float32)
    out_ref[...] = o + b3_ref[...]


BM = 1024


@jax.jit
def _mlp(emb, xc_pad, w1a, w1b, b1, w2p, b2p, w3p, b3p):
    grid = (B // BM,)
    return pl.pallas_call(
        _mlp_body,
        grid=grid,
        in_specs=[
            pl.BlockSpec((BM, F * D), lambda i: (i, 0)),
            pl.BlockSpec((BM, 128), lambda i: (i, 0)),
            pl.BlockSpec((F * D, H1), lambda i: (0, 0)),
            pl.BlockSpec((128, H1), lambda i: (0, 0)),
            pl.BlockSpec((1, H1), lambda i: (0, 0)),
            pl.BlockSpec((H1, 128), lambda i: (0, 0)),
            pl.BlockSpec((1, 128), lambda i: (0, 0)),
            pl.BlockSpec((128, 128), lambda i: (0, 0)),
            pl.BlockSpec((1, 128), lambda i: (0, 0)),
        ],
        out_specs=pl.BlockSpec((BM, 128), lambda i: (i, 0)),
        out_shape=jax.ShapeDtypeStruct((B, 128), jnp.float32),
    )(emb, xc_pad, w1a, w1b, b1, w2p, b2p, w3p, b3p)


def kernel(x_cat, x_cont, tables, W1, b1, W2, b2, W3, b3):
    table_t = tables.transpose(0, 2, 1)
    table_rows = _tc_transpose(table_t).reshape(FB * NXB * XB * 4, D)

    x = x_cat.astype(jnp.int32)
    farr = jnp.arange(F, dtype=jnp.int32)
    fblk = (farr // 4 * (NXB * XB * 4))[None, :]
    fk = (farr % 4)[None, :]
    flat_idx = fblk + (x // XB) * (XB * 4) + (x % XB) * 4 + fk
    idx2d = flat_idx.reshape((B * F) // IDX_ROW, IDX_ROW)

    emb = _sc_gather(table_rows, idx2d).reshape(B, F * D)

    xc_pad = jnp.pad(x_cont, ((0, 0), (0, 128 - C)))
    w1a = W1[:F * D]
    w1b = jnp.pad(W1[F * D:], ((0, 128 - C), (0, 0)))
    w2p = jnp.pad(W2, ((0, 0), (0, 128 - H2)))
    b2p = jnp.pad(b2, (0, 128 - H2)).reshape(1, 128)
    w3p = jnp.pad(W3, ((0, 128 - H2), (0, 128 - NCLS)))
    b3p = jnp.pad(b3, (0, 128 - NCLS)).reshape(1, 128)

    out = _mlp(emb, xc_pad, w1a, w1b, b1.reshape(1, H1), w2p, b2p, w3p, b3p)
    return out[:, :NCLS]
